# baseline (speedup 1.0000x reference)
import functools

import jax
import jax.numpy as jnp
from jax import lax
from jax.experimental import pallas as pl
from jax.experimental.pallas import tpu as pltpu
from jax.experimental.pallas import tpu_sc as plsc

N = 10000
E = 320000
F = 128
NG = 16
NP = 10112
DUMMY = 10016
R = 3 * NP
EPP = 327680
CH = 128
NT = 16
NC = 2
GPP = 80
DIN = 24
BR = 2528
RB = R // BR


def _sc_deg_body(dst4, zeros8, ones8, deg_out, acc, idxv, onesv):
    cid = lax.axis_index("c")
    sid = lax.axis_index("s")
    rpt = R // NT
    pltpu.sync_copy(zeros8.at[pl.ds(sid * rpt, rpt)], acc.at[pl.ds(sid * rpt, rpt)])
    pltpu.sync_copy(ones8, onesv)
    plsc.subcore_barrier()

    def outer(o, carry):
        pltpu.sync_copy(dst4.at[cid, sid, pl.ds(o * DIN, DIN)], idxv)
        for j in range(DIN):
            pltpu.sync_copy(onesv, acc.at[idxv.at[j]], add=True)
        return carry

    lax.fori_loop(0, EPP * 3 // (NC * NT * CH * DIN), outer, 0)
    plsc.subcore_barrier()
    pltpu.sync_copy(acc.at[pl.ds(sid * rpt, rpt)],
                    deg_out.at[cid, pl.ds(sid * rpt, rpt)])


def _sc_scatter_body(h2, src_i, dst_i, zeros, out_hbm, acc, srcv, dstv,
                     buf0, buf1, gs0, gs1, ss0, ss1):
    cid = lax.axis_index("c")
    sid = lax.axis_index("s")
    rpt = NP // NT
    bufs = (buf0, buf1)
    gsems = (gs0, gs1)
    ssems = (ss0, ss1)

    for p in range(3):
        @pl.when(cid == 0)
        def _init_h():
            pltpu.sync_copy(h2.at[pl.ds(p * NP + sid * rpt, rpt)],
                            acc.at[pl.ds(sid * rpt, rpt)])

        @pl.when(cid != 0)
        def _init_z():
            pltpu.sync_copy(zeros.at[pl.ds(sid * rpt, rpt)],
                            acc.at[pl.ds(sid * rpt, rpt)])

        plsc.subcore_barrier()

        def outer(o, carry):
            pltpu.sync_copy(src_i.at[p, cid, sid, pl.ds(o * 4, 4)], srcv)
            pltpu.sync_copy(dst_i.at[p, cid, sid, pl.ds(o * 4, 4)], dstv)
            for i in range(4):
                b = i % 2
                if i < 2:
                    @pl.when(o > 0)
                    def _drain():
                        pltpu.make_async_copy(
                            h2.at[pl.ds(0, CH)], bufs[b], ssems[b]).wait()
                else:
                    pltpu.make_async_copy(
                        h2.at[pl.ds(0, CH)], bufs[b], ssems[b]).wait()
                pltpu.async_copy(h2.at[srcv.at[i]], bufs[b], gsems[b]).wait()
                pltpu.async_copy(bufs[b], acc.at[dstv.at[i]], ssems[b],
                                 add=True)
            return carry

        lax.fori_loop(0, GPP // 4, outer, 0)
        for b in range(2):
            pltpu.make_async_copy(h2.at[pl.ds(0, CH)], bufs[b], ssems[b]).wait()
        plsc.subcore_barrier()
        pltpu.sync_copy(acc.at[pl.ds(sid * rpt, rpt)],
                        out_hbm.at[cid, p, pl.ds(sid * rpt, rpt)])
        plsc.subcore_barrier()


@functools.cache
def _sc_calls():
    mesh = plsc.VectorSubcoreMesh(core_axis_name="c", subcore_axis_name="s")
    cp = pltpu.CompilerParams(use_tc_tiling_on_sc=False)
    deg_call = pl.kernel(
        _sc_deg_body,
        out_type=jax.ShapeDtypeStruct((NC, R, 8), jnp.float32),
        mesh=mesh,
        compiler_params=cp,
        scratch_types=[
            pltpu.VMEM_SHARED((R, 8), jnp.float32),
            pltpu.VMEM((DIN, CH), jnp.int32),
            pltpu.VMEM((CH, 8), jnp.float32),
        ],
    )
    scat_call = pl.kernel(
        _sc_scatter_body,
        out_type=jax.ShapeDtypeStruct((NC, 3, NP, F), jnp.float32),
        mesh=mesh,
        compiler_params=cp,
        scratch_types=[
            pltpu.VMEM_SHARED((NP, F), jnp.float32),
            pltpu.VMEM((4, CH), jnp.int32),
            pltpu.VMEM((4, CH), jnp.int32),
            pltpu.VMEM((CH, F), jnp.float32),
            pltpu.VMEM((CH, F), jnp.float32),
            pltpu.SemaphoreType.DMA,
            pltpu.SemaphoreType.DMA,
            pltpu.SemaphoreType.DMA,
            pltpu.SemaphoreType.DMA,
        ],
    )
    return deg_call, scat_call


def _dinv_of(deg_ref):
    dtot = deg_ref[0, :, 0] + deg_ref[1, :, 0] + 1.0
    return lax.rsqrt(dtot)


def _tc_l1_body(x_ref, deg_ref, w_ref, out_ref):
    dinv = _dinv_of(deg_ref)
    h = jnp.dot(x_ref[...], w_ref[...], preferred_element_type=jnp.float32)
    out_ref[...] = h * dinv[:, None]


def _tc_mid_body(acc_ref, deg_ref, w_ref, b_ref, out_ref):
    dinv = _dinv_of(deg_ref)
    full = acc_ref[0, 0] + acc_ref[1, 0]
    z = jnp.maximum(full * dinv[:, None] + b_ref[...], 0.0)
    h = jnp.dot(z, w_ref[...], preferred_element_type=jnp.float32)
    out_ref[...] = h * dinv[:, None]


def _tc_pool_body(acc_ref, deg_ref, b_ref, seg_ref,
                  pooled_ref, cnt_ref, pacc, cacc):
    i = pl.program_id(0)

    @pl.when(i == 0)
    def _init():
        pacc[...] = jnp.zeros_like(pacc)
        cacc[...] = jnp.zeros_like(cacc)

    dinv = _dinv_of(deg_ref)
    full = acc_ref[0, 0] + acc_ref[1, 0]
    outc = full * dinv[:, None] + b_ref[...]
    seg = seg_ref[:, 0]
    cols = lax.broadcasted_iota(jnp.int32, (BR, 64), 1)
    p = (seg[:, None] == cols).astype(jnp.float32)
    pacc[...] += lax.dot_general(p, outc, (((0,), (0,)), ((), ())),
                                 preferred_element_type=jnp.float32)
    cacc[...] += jnp.broadcast_to(jnp.sum(p, axis=0)[:, None], (64, F))

    @pl.when(i == RB - 1)
    def _fin():
        pooled_ref[...] = pacc[...]
        cnt_ref[...] = cacc[...]


def _tc_head_body(pooled_ref, cnt_ref, l0w_ref, l0b_ref, lw_ref, lb_ref,
                  e0_ref, e1_ref, e2_ref, corr_ref, sp_ref, sn_ref, cs_ref):
    mean = pooled_ref[...] / jnp.maximum(cnt_ref[...], 1.0)
    e = jnp.dot(mean, l0w_ref[...], preferred_element_type=jnp.float32) + l0b_ref[...]
    e0 = e[0:16]
    e1 = e[16:32]
    e2 = e[32:48]
    e0_ref[...] = e0
    e1_ref[...] = e1
    e2_ref[...] = e2
    dp = jnp.sqrt(jnp.sum((e0 - e1 + 1e-6) ** 2, axis=1, keepdims=True))
    dn = jnp.sqrt(jnp.sum((e0 - e2 + 1e-6) ** 2, axis=1, keepdims=True))
    lw = lw_ref[...]
    y1 = (jnp.sum(e0 * lw[:, :64], axis=1, keepdims=True)
          + jnp.sum(e1 * lw[:, 64:], axis=1, keepdims=True) + lb_ref[...])
    y2 = (jnp.sum(e0 * lw[:, :64], axis=1, keepdims=True)
          + jnp.sum(e2 * lw[:, 64:], axis=1, keepdims=True) + lb_ref[...])
    sp = jax.nn.sigmoid(y1)
    sn = jax.nn.sigmoid(y2)
    sp_ref[...] = sp
    sn_ref[...] = sn
    corr_ref[...] = jnp.sum((dn - dp > 0).astype(jnp.int32), axis=(0, 1),
                            keepdims=True)
    cs_ref[...] = jnp.sum((sp - sn > 0).astype(jnp.int32), axis=(0, 1),
                          keepdims=True)


def _tc_l1(x, degacc, w0):
    return pl.pallas_call(
        _tc_l1_body,
        grid=(RB,),
        in_specs=[
            pl.BlockSpec((BR, F), lambda i: (i, 0)),
            pl.BlockSpec((NC, BR, 8), lambda i: (0, i, 0)),
            pl.BlockSpec((F, F), lambda i: (0, 0)),
        ],
        out_specs=pl.BlockSpec((BR, F), lambda i: (i, 0)),
        out_shape=jax.ShapeDtypeStruct((R, F), jnp.float32),
    )(x, degacc, w0)


def _tc_mid(acc, degacc, w, b):
    return pl.pallas_call(
        _tc_mid_body,
        grid=(RB,),
        in_specs=[
            pl.BlockSpec((NC, 1, BR, F), lambda i: (0, i // 4, i % 4, 0)),
            pl.BlockSpec((NC, BR, 8), lambda i: (0, i, 0)),
            pl.BlockSpec((F, F), lambda i: (0, 0)),
            pl.BlockSpec((1, F), lambda i: (0, 0)),
        ],
        out_specs=pl.BlockSpec((BR, F), lambda i: (i, 0)),
        out_shape=jax.ShapeDtypeStruct((R, F), jnp.float32),
    )(acc, degacc, w, b)


def _tc_pool(acc, degacc, b, seg8):
    return pl.pallas_call(
        _tc_pool_body,
        grid=(RB,),
        in_specs=[
            pl.BlockSpec((NC, 1, BR, F), lambda i: (0, i // 4, i % 4, 0)),
            pl.BlockSpec((NC, BR, 8), lambda i: (0, i, 0)),
            pl.BlockSpec((1, F), lambda i: (0, 0)),
            pl.BlockSpec((BR, 8), lambda i: (i, 0)),
        ],
        out_specs=[
            pl.BlockSpec((64, F), lambda i: (0, 0)),
            pl.BlockSpec((64, F), lambda i: (0, 0)),
        ],
        out_shape=[
            jax.ShapeDtypeStruct((64, F), jnp.float32),
            jax.ShapeDtypeStruct((64, F), jnp.float32),
        ],
        scratch_shapes=[
            pltpu.VMEM((64, F), jnp.float32),
            pltpu.VMEM((64, F), jnp.float32),
        ],
    )(acc, degacc, b, seg8)


def _tc_head(pooled, cnt, l0w, l0b, lw, lb):
    return pl.pallas_call(
        _tc_head_body,
        out_shape=[
            jax.ShapeDtypeStruct((NG, 64), jnp.float32),
            jax.ShapeDtypeStruct((NG, 64), jnp.float32),
            jax.ShapeDtypeStruct((NG, 64), jnp.float32),
            jax.ShapeDtypeStruct((1, 1), jnp.int32),
            jax.ShapeDtypeStruct((NG, 1), jnp.float32),
            jax.ShapeDtypeStruct((NG, 1), jnp.float32),
            jax.ShapeDtypeStruct((1, 1), jnp.int32),
        ],
    )(pooled, cnt, l0w, l0b, lw, lb)


def kernel(x0, edge_index0, batch0, x1, edge_index1, batch1,
           x2, edge_index2, batch2, params):
    xs = (x0, x1, x2)
    eis = (edge_index0, edge_index1, edge_index2)
    bs = (batch0, batch1, batch2)

    zpad = jnp.zeros((NP - N, F), jnp.float32)
    x_all = jnp.concatenate([jnp.concatenate([x, zpad]) for x in xs])
    epad = EPP - E
    srcs, dsts, dstg = [], [], []
    for p in range(3):
        s = jnp.concatenate([eis[p][0] + p * NP, jnp.zeros((epad,), jnp.int32)])
        d = jnp.concatenate([eis[p][1], jnp.full((epad,), DUMMY, jnp.int32)])
        srcs.append(s.reshape(NC, NT, GPP, CH))
        dsts.append(d.reshape(NC, NT, GPP, CH))
        dstg.append(d + p * NP)
    src_i = jnp.stack(srcs)
    dst_i = jnp.stack(dsts)
    dst4 = jnp.concatenate(dstg).reshape(NC, NT, 3 * EPP // (NC * NT * CH), CH)
    segpad = jnp.full((NP - N,), 48, jnp.int32)
    seg = jnp.concatenate(
        [jnp.concatenate([bs[p] + NG * p, segpad]) for p in range(3)])
    seg8 = jnp.broadcast_to(seg[:, None], (R, 8))

    zeros8 = jnp.zeros((R, 8), jnp.float32)
    zerosf = jnp.zeros((NP, F), jnp.float32)
    ones8 = jnp.ones((CH, 8), jnp.float32)

    w = params["conv_W"]
    cb = params["conv_b"]
    b0, b1, b2 = (cb[i].reshape(1, F) for i in range(3))

    deg_call, scat_call = _sc_calls()
    degacc = deg_call(dst4, zeros8, ones8)
    h1p = _tc_l1(x_all, degacc, w[0])
    a1 = scat_call(h1p, src_i, dst_i, zerosf)
    h2p = _tc_mid(a1, degacc, w[1], b0)
    a2 = scat_call(h2p, src_i, dst_i, zerosf)
    h3p = _tc_mid(a2, degacc, w[2], b1)
    a3 = scat_call(h3p, src_i, dst_i, zerosf)
    pooled, cnt = _tc_pool(a3, degacc, b2, seg8)
    e0, e1, e2, corr, sp, sn, cs = _tc_head(
        pooled, cnt, params["lin0_W"], params["lin0_b"].reshape(1, 64),
        params["lin_W"].reshape(1, F), params["lin_b"].reshape(1, 1))
    return (e0, e1, e2, corr.reshape(1), sp, sn, cs.reshape(1))


# speedup vs baseline: 1.4582x; 1.4582x over previous
import functools

import jax
import jax.numpy as jnp
from jax import lax
from jax.experimental import pallas as pl
from jax.experimental.pallas import tpu as pltpu
from jax.experimental.pallas import tpu_sc as plsc

N = 10000
E = 320000
F = 128
FH = 64
NG = 16
NP = 10112
DUMMY = 10016
R = 3 * NP
EPP = 327680
CH = 128
NT = 16
NC = 2
GPP = 160
NBUF = 4
DIN = 24
BR = 2528
RB = R // BR


def _sc_deg_body(dst4, zeros8, ones8, deg_out, acc, idxv, onesv):
    cid = lax.axis_index("c")
    sid = lax.axis_index("s")
    rpt = R // NT
    pltpu.sync_copy(zeros8.at[pl.ds(sid * rpt, rpt)], acc.at[pl.ds(sid * rpt, rpt)])
    pltpu.sync_copy(ones8, onesv)
    plsc.subcore_barrier()

    def outer(o, carry):
        pltpu.sync_copy(dst4.at[cid, sid, pl.ds(o * DIN, DIN)], idxv)
        for j in range(DIN):
            pltpu.sync_copy(onesv, acc.at[idxv.at[j]], add=True)
        return carry

    lax.fori_loop(0, EPP * 3 // (NC * NT * CH * DIN), outer, 0)
    plsc.subcore_barrier()
    pltpu.sync_copy(acc.at[pl.ds(sid * rpt, rpt)],
                    deg_out.at[cid, pl.ds(sid * rpt, rpt)])


def _sc_scatter_body(h2, src_i, dst_i, out_hbm, acc, srcv, dstv,
                     b0, b1, b2, b3, g0, g1, g2, g3, s0, s1, s2, s3):
    cid = lax.axis_index("c")
    sid = lax.axis_index("s")
    rpt = NP // NT
    bufs = (b0, b1, b2, b3)
    gsems = (g0, g1, g2, g3)
    ssems = (s0, s1, s2, s3)
    hme = h2.at[cid]

    for p in range(3):
        pltpu.sync_copy(hme.at[pl.ds(p * NP + sid * rpt, rpt)],
                        acc.at[pl.ds(sid * rpt, rpt)])
        plsc.subcore_barrier()

        def outer(o, carry):
            pltpu.sync_copy(src_i.at[p, sid, pl.ds(o * NBUF, NBUF)], srcv)
            pltpu.sync_copy(dst_i.at[p, sid, pl.ds(o * NBUF, NBUF)], dstv)
            for i in range(NBUF):
                @pl.when(o > 0)
                def _drain():
                    pltpu.make_async_copy(
                        hme.at[pl.ds(0, CH)], bufs[i], ssems[i]).wait()

            gds = [pltpu.async_copy(hme.at[srcv.at[i]], bufs[i], gsems[i])
                   for i in range(NBUF)]
            for i in range(NBUF):
                gds[i].wait()
                pltpu.async_copy(bufs[i], acc.at[dstv.at[i]], ssems[i],
                                 add=True)
            return carry

        lax.fori_loop(0, GPP // NBUF, outer, 0)
        for i in range(NBUF):
            pltpu.make_async_copy(hme.at[pl.ds(0, CH)], bufs[i], ssems[i]).wait()
        plsc.subcore_barrier()
        pltpu.sync_copy(acc.at[pl.ds(sid * rpt, rpt)],
                        out_hbm.at[cid, p, pl.ds(sid * rpt, rpt)])
        plsc.subcore_barrier()


@functools.cache
def _sc_calls():
    mesh = plsc.VectorSubcoreMesh(core_axis_name="c", subcore_axis_name="s")
    cp = pltpu.CompilerParams(use_tc_tiling_on_sc=False)
    deg_call = pl.kernel(
        _sc_deg_body,
        out_type=jax.ShapeDtypeStruct((NC, R, 8), jnp.float32),
        mesh=mesh,
        compiler_params=cp,
        scratch_types=[
            pltpu.VMEM_SHARED((R, 8), jnp.float32),
            pltpu.VMEM((DIN, CH), jnp.int32),
            pltpu.VMEM((CH, 8), jnp.float32),
        ],
    )
    scat_call = pl.kernel(
        _sc_scatter_body,
        out_type=jax.ShapeDtypeStruct((NC, 3, NP, FH), jnp.float32),
        mesh=mesh,
        compiler_params=cp,
        scratch_types=(
            [pltpu.VMEM_SHARED((NP, FH), jnp.float32),
             pltpu.VMEM((NBUF, CH), jnp.int32),
             pltpu.VMEM((NBUF, CH), jnp.int32)]
            + [pltpu.VMEM((CH, FH), jnp.float32)] * NBUF
            + [pltpu.SemaphoreType.DMA] * (2 * NBUF)
        ),
    )
    return deg_call, scat_call


def _dinv_of(deg_ref):
    dtot = deg_ref[0, :, 0] + deg_ref[1, :, 0] + 1.0
    return lax.rsqrt(dtot)


def _tc_l1_body(x_ref, deg_ref, w_ref, out_ref):
    dinv = _dinv_of(deg_ref)
    h = jnp.dot(x_ref[...], w_ref[...], preferred_element_type=jnp.float32)
    hp = h * dinv[:, None]
    out_ref[0] = hp[:, :FH]
    out_ref[1] = hp[:, FH:]


def _tc_mid_body(acc_ref, deg_ref, w_ref, b_ref, out_ref):
    dinv = _dinv_of(deg_ref)
    full = jnp.concatenate([acc_ref[0, 0], acc_ref[1, 0]], axis=1)
    z = jnp.maximum(full * dinv[:, None] + b_ref[...], 0.0)
    h = jnp.dot(z, w_ref[...], preferred_element_type=jnp.float32)
    hp = h * dinv[:, None]
    out_ref[0] = hp[:, :FH]
    out_ref[1] = hp[:, FH:]


def _tc_pool_body(acc_ref, deg_ref, b_ref, seg_ref,
                  pooled_ref, cnt_ref, pacc, cacc):
    i = pl.program_id(0)

    @pl.when(i == 0)
    def _init():
        pacc[...] = jnp.zeros_like(pacc)
        cacc[...] = jnp.zeros_like(cacc)

    dinv = _dinv_of(deg_ref)
    full = jnp.concatenate([acc_ref[0, 0], acc_ref[1, 0]], axis=1)
    outc = full * dinv[:, None] + b_ref[...]
    seg = seg_ref[:, 0]
    cols = lax.broadcasted_iota(jnp.int32, (BR, 64), 1)
    p = (seg[:, None] == cols).astype(jnp.float32)
    pacc[...] += lax.dot_general(p, outc, (((0,), (0,)), ((), ())),
                                 preferred_element_type=jnp.float32)
    cacc[...] += jnp.broadcast_to(jnp.sum(p, axis=0)[:, None], (64, F))

    @pl.when(i == RB - 1)
    def _fin():
        pooled_ref[...] = pacc[...]
        cnt_ref[...] = cacc[...]


def _tc_head_body(pooled_ref, cnt_ref, l0w_ref, l0b_ref, lw_ref, lb_ref,
                  e0_ref, e1_ref, e2_ref, corr_ref, sp_ref, sn_ref, cs_ref):
    mean = pooled_ref[...] / jnp.maximum(cnt_ref[...], 1.0)
    e = jnp.dot(mean, l0w_ref[...], preferred_element_type=jnp.float32) + l0b_ref[...]
    e0 = e[0:16]
    e1 = e[16:32]
    e2 = e[32:48]
    e0_ref[...] = e0
    e1_ref[...] = e1
    e2_ref[...] = e2
    dp = jnp.sqrt(jnp.sum((e0 - e1 + 1e-6) ** 2, axis=1, keepdims=True))
    dn = jnp.sqrt(jnp.sum((e0 - e2 + 1e-6) ** 2, axis=1, keepdims=True))
    lw = lw_ref[...]
    y1 = (jnp.sum(e0 * lw[:, :64], axis=1, keepdims=True)
          + jnp.sum(e1 * lw[:, 64:], axis=1, keepdims=True) + lb_ref[...])
    y2 = (jnp.sum(e0 * lw[:, :64], axis=1, keepdims=True)
          + jnp.sum(e2 * lw[:, 64:], axis=1, keepdims=True) + lb_ref[...])
    sp = jax.nn.sigmoid(y1)
    sn = jax.nn.sigmoid(y2)
    sp_ref[...] = sp
    sn_ref[...] = sn
    corr_ref[...] = jnp.sum((dn - dp > 0).astype(jnp.int32), axis=(0, 1),
                            keepdims=True)
    cs_ref[...] = jnp.sum((sp - sn > 0).astype(jnp.int32), axis=(0, 1),
                          keepdims=True)


def _tc_l1(x, degacc, w0):
    return pl.pallas_call(
        _tc_l1_body,
        grid=(RB,),
        in_specs=[
            pl.BlockSpec((BR, F), lambda i: (i, 0)),
            pl.BlockSpec((NC, BR, 8), lambda i: (0, i, 0)),
            pl.BlockSpec((F, F), lambda i: (0, 0)),
        ],
        out_specs=pl.BlockSpec((NC, BR, FH), lambda i: (0, i, 0)),
        out_shape=jax.ShapeDtypeStruct((NC, R, FH), jnp.float32),
    )(x, degacc, w0)


def _tc_mid(acc, degacc, w, b):
    return pl.pallas_call(
        _tc_mid_body,
        grid=(RB,),
        in_specs=[
            pl.BlockSpec((NC, 1, BR, FH), lambda i: (0, i // 4, i % 4, 0)),
            pl.BlockSpec((NC, BR, 8), lambda i: (0, i, 0)),
            pl.BlockSpec((F, F), lambda i: (0, 0)),
            pl.BlockSpec((1, F), lambda i: (0, 0)),
        ],
        out_specs=pl.BlockSpec((NC, BR, FH), lambda i: (0, i, 0)),
        out_shape=jax.ShapeDtypeStruct((NC, R, FH), jnp.float32),
    )(acc, degacc, w, b)


def _tc_pool(acc, degacc, b, seg8):
    return pl.pallas_call(
        _tc_pool_body,
        grid=(RB,),
        in_specs=[
            pl.BlockSpec((NC, 1, BR, FH), lambda i: (0, i // 4, i % 4, 0)),
            pl.BlockSpec((NC, BR, 8), lambda i: (0, i, 0)),
            pl.BlockSpec((1, F), lambda i: (0, 0)),
            pl.BlockSpec((BR, 8), lambda i: (i, 0)),
        ],
        out_specs=[
            pl.BlockSpec((64, F), lambda i: (0, 0)),
            pl.BlockSpec((64, F), lambda i: (0, 0)),
        ],
        out_shape=[
            jax.ShapeDtypeStruct((64, F), jnp.float32),
            jax.ShapeDtypeStruct((64, F), jnp.float32),
        ],
        scratch_shapes=[
            pltpu.VMEM((64, F), jnp.float32),
            pltpu.VMEM((64, F), jnp.float32),
        ],
    )(acc, degacc, b, seg8)


def _tc_head(pooled, cnt, l0w, l0b, lw, lb):
    return pl.pallas_call(
        _tc_head_body,
        out_shape=[
            jax.ShapeDtypeStruct((NG, 64), jnp.float32),
            jax.ShapeDtypeStruct((NG, 64), jnp.float32),
            jax.ShapeDtypeStruct((NG, 64), jnp.float32),
            jax.ShapeDtypeStruct((1, 1), jnp.int32),
            jax.ShapeDtypeStruct((NG, 1), jnp.float32),
            jax.ShapeDtypeStruct((NG, 1), jnp.float32),
            jax.ShapeDtypeStruct((1, 1), jnp.int32),
        ],
    )(pooled, cnt, l0w, l0b, lw, lb)


def kernel(x0, edge_index0, batch0, x1, edge_index1, batch1,
           x2, edge_index2, batch2, params):
    xs = (x0, x1, x2)
    eis = (edge_index0, edge_index1, edge_index2)
    bs = (batch0, batch1, batch2)

    zpad = jnp.zeros((NP - N, F), jnp.float32)
    x_all = jnp.concatenate([jnp.concatenate([x, zpad]) for x in xs])
    epad = EPP - E
    srcs, dsts, dstg = [], [], []
    for p in range(3):
        s = jnp.concatenate([eis[p][0] + p * NP, jnp.zeros((epad,), jnp.int32)])
        d = jnp.concatenate([eis[p][1], jnp.full((epad,), DUMMY, jnp.int32)])
        srcs.append(s.reshape(NT, GPP, CH))
        dsts.append(d.reshape(NT, GPP, CH))
        dstg.append(d + p * NP)
    src_i = jnp.stack(srcs)
    dst_i = jnp.stack(dsts)
    dst4 = jnp.concatenate(dstg).reshape(NC, NT, 3 * EPP // (NC * NT * CH), CH)
    segpad = jnp.full((NP - N,), 48, jnp.int32)
    seg = jnp.concatenate(
        [jnp.concatenate([bs[p] + NG * p, segpad]) for p in range(3)])
    seg8 = jnp.broadcast_to(seg[:, None], (R, 8))

    zeros8 = jnp.zeros((R, 8), jnp.float32)
    ones8 = jnp.ones((CH, 8), jnp.float32)

    w = params["conv_W"]
    cb = params["conv_b"]
    b0, b1, b2 = (cb[i].reshape(1, F) for i in range(3))

    deg_call, scat_call = _sc_calls()
    degacc = deg_call(dst4, zeros8, ones8)
    h1p = _tc_l1(x_all, degacc, w[0])
    a1 = scat_call(h1p, src_i, dst_i)
    h2p = _tc_mid(a1, degacc, w[1], b0)
    a2 = scat_call(h2p, src_i, dst_i)
    h3p = _tc_mid(a2, degacc, w[2], b1)
    a3 = scat_call(h3p, src_i, dst_i)
    pooled, cnt = _tc_pool(a3, degacc, b2, seg8)
    e0, e1, e2, corr, sp, sn, cs = _tc_head(
        pooled, cnt, params["lin0_W"], params["lin0_b"].reshape(1, 64),
        params["lin_W"].reshape(1, F), params["lin_b"].reshape(1, 1))
    return (e0, e1, e2, corr.reshape(1), sp, sn, cs.reshape(1))


# speedup vs baseline: 1.5436x; 1.0586x over previous
import functools

import jax
import jax.numpy as jnp
from jax import lax
from jax.experimental import pallas as pl
from jax.experimental.pallas import tpu as pltpu
from jax.experimental.pallas import tpu_sc as plsc

N = 10000
E = 320000
F = 128
FH = 64
NG = 16
NP = 10112
DUMMY = 10016
R = 3 * NP
EPP = 327680
CH = 128
NT = 16
NC = 2
GPP = 160
NBUF = 8
DIN = 24
BR = 2528
RB = R // BR


def _sc_deg_body(dst4, zeros8, ones8, deg_out, acc, idxv, onesv):
    cid = lax.axis_index("c")
    sid = lax.axis_index("s")
    rpt = R // NT
    pltpu.sync_copy(zeros8.at[pl.ds(sid * rpt, rpt)], acc.at[pl.ds(sid * rpt, rpt)])
    pltpu.sync_copy(ones8, onesv)
    plsc.subcore_barrier()

    def outer(o, carry):
        pltpu.sync_copy(dst4.at[cid, sid, pl.ds(o * DIN, DIN)], idxv)
        for j in range(DIN):
            pltpu.sync_copy(onesv, acc.at[idxv.at[j]], add=True)
        return carry

    lax.fori_loop(0, EPP * 3 // (NC * NT * CH * DIN), outer, 0)
    plsc.subcore_barrier()
    pltpu.sync_copy(acc.at[pl.ds(sid * rpt, rpt)],
                    deg_out.at[cid, pl.ds(sid * rpt, rpt)])


def _sc_scatter_body(h2, src_i, dst_i, out_hbm, acc, srcv, dstv, *rest):
    cid = lax.axis_index("c")
    sid = lax.axis_index("s")
    rpt = NP // NT
    bufs = rest[:NBUF]
    gsems = rest[NBUF:2 * NBUF]
    ssems = rest[2 * NBUF:]
    hme = h2.at[cid]

    for p in range(3):
        pltpu.sync_copy(hme.at[pl.ds(p * NP + sid * rpt, rpt)],
                        acc.at[pl.ds(sid * rpt, rpt)])
        plsc.subcore_barrier()

        def outer(o, carry):
            pltpu.sync_copy(src_i.at[p, sid, pl.ds(o * NBUF, NBUF)], srcv)
            pltpu.sync_copy(dst_i.at[p, sid, pl.ds(o * NBUF, NBUF)], dstv)
            for i in range(NBUF):
                @pl.when(o > 0)
                def _drain():
                    pltpu.make_async_copy(
                        hme.at[pl.ds(0, CH)], bufs[i], ssems[i]).wait()

            gds = [pltpu.async_copy(hme.at[srcv.at[i]], bufs[i], gsems[i])
                   for i in range(NBUF)]
            for i in range(NBUF):
                gds[i].wait()
                pltpu.async_copy(bufs[i], acc.at[dstv.at[i]], ssems[i],
                                 add=True)
            return carry

        lax.fori_loop(0, GPP // NBUF, outer, 0)
        for i in range(NBUF):
            pltpu.make_async_copy(hme.at[pl.ds(0, CH)], bufs[i], ssems[i]).wait()
        plsc.subcore_barrier()
        pltpu.sync_copy(acc.at[pl.ds(sid * rpt, rpt)],
                        out_hbm.at[cid, p, pl.ds(sid * rpt, rpt)])
        plsc.subcore_barrier()


@functools.cache
def _sc_calls():
    mesh = plsc.VectorSubcoreMesh(core_axis_name="c", subcore_axis_name="s")
    cp = pltpu.CompilerParams(use_tc_tiling_on_sc=False)
    deg_call = pl.kernel(
        _sc_deg_body,
        out_type=jax.ShapeDtypeStruct((NC, R, 8), jnp.float32),
        mesh=mesh,
        compiler_params=cp,
        scratch_types=[
            pltpu.VMEM_SHARED((R, 8), jnp.float32),
            pltpu.VMEM((DIN, CH), jnp.int32),
            pltpu.VMEM((CH, 8), jnp.float32),
        ],
    )
    scat_call = pl.kernel(
        _sc_scatter_body,
        out_type=jax.ShapeDtypeStruct((NC, 3, NP, FH), jnp.float32),
        mesh=mesh,
        compiler_params=cp,
        scratch_types=(
            [pltpu.VMEM_SHARED((NP, FH), jnp.float32),
             pltpu.VMEM((NBUF, CH), jnp.int32),
             pltpu.VMEM((NBUF, CH), jnp.int32)]
            + [pltpu.VMEM((CH, FH), jnp.float32)] * NBUF
            + [pltpu.SemaphoreType.DMA] * (2 * NBUF)
        ),
    )
    return deg_call, scat_call


def _dinv_of(deg_ref):
    dtot = deg_ref[0, :, 0] + deg_ref[1, :, 0] + 1.0
    return lax.rsqrt(dtot)


def _tc_l1_body(x_ref, deg_ref, w_ref, out_ref):
    dinv = _dinv_of(deg_ref)
    h = jnp.dot(x_ref[...], w_ref[...], preferred_element_type=jnp.float32)
    hp = h * dinv[:, None]
    out_ref[0] = hp[:, :FH]
    out_ref[1] = hp[:, FH:]


def _tc_mid_body(acc_ref, deg_ref, w_ref, b_ref, out_ref):
    dinv = _dinv_of(deg_ref)
    full = jnp.concatenate([acc_ref[0, 0], acc_ref[1, 0]], axis=1)
    z = jnp.maximum(full * dinv[:, None] + b_ref[...], 0.0)
    h = jnp.dot(z, w_ref[...], preferred_element_type=jnp.float32)
    hp = h * dinv[:, None]
    out_ref[0] = hp[:, :FH]
    out_ref[1] = hp[:, FH:]


def _tc_pool_body(acc_ref, deg_ref, b_ref, seg_ref,
                  pooled_ref, cnt_ref, pacc, cacc):
    i = pl.program_id(0)

    @pl.when(i == 0)
    def _init():
        pacc[...] = jnp.zeros_like(pacc)
        cacc[...] = jnp.zeros_like(cacc)

    dinv = _dinv_of(deg_ref)
    full = jnp.concatenate([acc_ref[0, 0], acc_ref[1, 0]], axis=1)
    outc = full * dinv[:, None] + b_ref[...]
    seg = seg_ref[:, 0]
    cols = lax.broadcasted_iota(jnp.int32, (BR, 64), 1)
    p = (seg[:, None] == cols).astype(jnp.float32)
    pacc[...] += lax.dot_general(p, outc, (((0,), (0,)), ((), ())),
                                 preferred_element_type=jnp.float32)
    cacc[...] += jnp.broadcast_to(jnp.sum(p, axis=0)[:, None], (64, F))

    @pl.when(i == RB - 1)
    def _fin():
        pooled_ref[...] = pacc[...]
        cnt_ref[...] = cacc[...]


def _tc_head_body(pooled_ref, cnt_ref, l0w_ref, l0b_ref, lw_ref, lb_ref,
                  e0_ref, e1_ref, e2_ref, corr_ref, sp_ref, sn_ref, cs_ref):
    mean = pooled_ref[...] / jnp.maximum(cnt_ref[...], 1.0)
    e = jnp.dot(mean, l0w_ref[...], preferred_element_type=jnp.float32) + l0b_ref[...]
    e0 = e[0:16]
    e1 = e[16:32]
    e2 = e[32:48]
    e0_ref[...] = e0
    e1_ref[...] = e1
    e2_ref[...] = e2
    dp = jnp.sqrt(jnp.sum((e0 - e1 + 1e-6) ** 2, axis=1, keepdims=True))
    dn = jnp.sqrt(jnp.sum((e0 - e2 + 1e-6) ** 2, axis=1, keepdims=True))
    lw = lw_ref[...]
    y1 = (jnp.sum(e0 * lw[:, :64], axis=1, keepdims=True)
          + jnp.sum(e1 * lw[:, 64:], axis=1, keepdims=True) + lb_ref[...])
    y2 = (jnp.sum(e0 * lw[:, :64], axis=1, keepdims=True)
          + jnp.sum(e2 * lw[:, 64:], axis=1, keepdims=True) + lb_ref[...])
    sp = jax.nn.sigmoid(y1)
    sn = jax.nn.sigmoid(y2)
    sp_ref[...] = sp
    sn_ref[...] = sn
    corr_ref[...] = jnp.sum((dn - dp > 0).astype(jnp.int32), axis=(0, 1),
                            keepdims=True)
    cs_ref[...] = jnp.sum((sp - sn > 0).astype(jnp.int32), axis=(0, 1),
                          keepdims=True)


def _tc_l1(x, degacc, w0):
    return pl.pallas_call(
        _tc_l1_body,
        grid=(RB,),
        in_specs=[
            pl.BlockSpec((BR, F), lambda i: (i, 0)),
            pl.BlockSpec((NC, BR, 8), lambda i: (0, i, 0)),
            pl.BlockSpec((F, F), lambda i: (0, 0)),
        ],
        out_specs=pl.BlockSpec((NC, BR, FH), lambda i: (0, i, 0)),
        out_shape=jax.ShapeDtypeStruct((NC, R, FH), jnp.float32),
    )(x, degacc, w0)


def _tc_mid(acc, degacc, w, b):
    return pl.pallas_call(
        _tc_mid_body,
        grid=(RB,),
        in_specs=[
            pl.BlockSpec((NC, 1, BR, FH), lambda i: (0, i // 4, i % 4, 0)),
            pl.BlockSpec((NC, BR, 8), lambda i: (0, i, 0)),
            pl.BlockSpec((F, F), lambda i: (0, 0)),
            pl.BlockSpec((1, F), lambda i: (0, 0)),
        ],
        out_specs=pl.BlockSpec((NC, BR, FH), lambda i: (0, i, 0)),
        out_shape=jax.ShapeDtypeStruct((NC, R, FH), jnp.float32),
    )(acc, degacc, w, b)


def _tc_pool(acc, degacc, b, seg8):
    return pl.pallas_call(
        _tc_pool_body,
        grid=(RB,),
        in_specs=[
            pl.BlockSpec((NC, 1, BR, FH), lambda i: (0, i // 4, i % 4, 0)),
            pl.BlockSpec((NC, BR, 8), lambda i: (0, i, 0)),
            pl.BlockSpec((1, F), lambda i: (0, 0)),
            pl.BlockSpec((BR, 8), lambda i: (i, 0)),
        ],
        out_specs=[
            pl.BlockSpec((64, F), lambda i: (0, 0)),
            pl.BlockSpec((64, F), lambda i: (0, 0)),
        ],
        out_shape=[
            jax.ShapeDtypeStruct((64, F), jnp.float32),
            jax.ShapeDtypeStruct((64, F), jnp.float32),
        ],
        scratch_shapes=[
            pltpu.VMEM((64, F), jnp.float32),
            pltpu.VMEM((64, F), jnp.float32),
        ],
    )(acc, degacc, b, seg8)


def _tc_head(pooled, cnt, l0w, l0b, lw, lb):
    return pl.pallas_call(
        _tc_head_body,
        out_shape=[
            jax.ShapeDtypeStruct((NG, 64), jnp.float32),
            jax.ShapeDtypeStruct((NG, 64), jnp.float32),
            jax.ShapeDtypeStruct((NG, 64), jnp.float32),
            jax.ShapeDtypeStruct((1, 1), jnp.int32),
            jax.ShapeDtypeStruct((NG, 1), jnp.float32),
            jax.ShapeDtypeStruct((NG, 1), jnp.float32),
            jax.ShapeDtypeStruct((1, 1), jnp.int32),
        ],
    )(pooled, cnt, l0w, l0b, lw, lb)


def kernel(x0, edge_index0, batch0, x1, edge_index1, batch1,
           x2, edge_index2, batch2, params):
    xs = (x0, x1, x2)
    eis = (edge_index0, edge_index1, edge_index2)
    bs = (batch0, batch1, batch2)

    zpad = jnp.zeros((NP - N, F), jnp.float32)
    x_all = jnp.concatenate([jnp.concatenate([x, zpad]) for x in xs])
    epad = EPP - E
    srcs, dsts, dstg = [], [], []
    for p in range(3):
        s = jnp.concatenate([eis[p][0] + p * NP, jnp.zeros((epad,), jnp.int32)])
        d = jnp.concatenate([eis[p][1], jnp.full((epad,), DUMMY, jnp.int32)])
        srcs.append(s.reshape(NT, GPP, CH))
        dsts.append(d.reshape(NT, GPP, CH))
        dstg.append(d + p * NP)
    src_i = jnp.stack(srcs)
    dst_i = jnp.stack(dsts)
    dst4 = jnp.concatenate(dstg).reshape(NC, NT, 3 * EPP // (NC * NT * CH), CH)
    segpad = jnp.full((NP - N,), 48, jnp.int32)
    seg = jnp.concatenate(
        [jnp.concatenate([bs[p] + NG * p, segpad]) for p in range(3)])
    seg8 = jnp.broadcast_to(seg[:, None], (R, 8))

    zeros8 = jnp.zeros((R, 8), jnp.float32)
    ones8 = jnp.ones((CH, 8), jnp.float32)

    w = params["conv_W"]
    cb = params["conv_b"]
    b0, b1, b2 = (cb[i].reshape(1, F) for i in range(3))

    deg_call, scat_call = _sc_calls()
    degacc = deg_call(dst4, zeros8, ones8)
    h1p = _tc_l1(x_all, degacc, w[0])
    a1 = scat_call(h1p, src_i, dst_i)
    h2p = _tc_mid(a1, degacc, w[1], b0)
    a2 = scat_call(h2p, src_i, dst_i)
    h3p = _tc_mid(a2, degacc, w[2], b1)
    a3 = scat_call(h3p, src_i, dst_i)
    pooled, cnt = _tc_pool(a3, degacc, b2, seg8)
    e0, e1, e2, corr, sp, sn, cs = _tc_head(
        pooled, cnt, params["lin0_W"], params["lin0_b"].reshape(1, 64),
        params["lin_W"].reshape(1, F), params["lin_b"].reshape(1, 1))
    return (e0, e1, e2, corr.reshape(1), sp, sn, cs.reshape(1))


# speedup vs baseline: 1.5437x; 1.0000x over previous
import functools

import jax
import jax.numpy as jnp
from jax import lax
from jax.experimental import pallas as pl
from jax.experimental.pallas import tpu as pltpu
from jax.experimental.pallas import tpu_sc as plsc

N = 10000
E = 320000
F = 128
FH = 64
NG = 16
NP = 10112
DUMMY = 10016
R = 3 * NP
EPP = 327680
CH = 128
NT = 16
NC = 2
GPP = 160
NBUF = 8
DIN = 24
BR = 2528
RB = R // BR


def _sc_deg_body(dst4, zeros8, ones8, deg_out, acc, idxv, onesv):
    cid = lax.axis_index("c")
    sid = lax.axis_index("s")
    rpt = R // NT
    pltpu.sync_copy(zeros8.at[pl.ds(sid * rpt, rpt)], acc.at[pl.ds(sid * rpt, rpt)])
    pltpu.sync_copy(ones8, onesv)
    plsc.subcore_barrier()

    def outer(o, carry):
        pltpu.sync_copy(dst4.at[cid, sid, pl.ds(o * DIN, DIN)], idxv)
        for j in range(DIN):
            pltpu.sync_copy(onesv, acc.at[idxv.at[j]], add=True)
        return carry

    lax.fori_loop(0, EPP * 3 // (NC * NT * CH * DIN), outer, 0)
    plsc.subcore_barrier()
    pltpu.sync_copy(acc.at[pl.ds(sid * rpt, rpt)],
                    deg_out.at[cid, pl.ds(sid * rpt, rpt)])


def _sc_scatter_body(h2, src_i, dst_i, out_hbm, acc, srcv, dstv, *rest):
    cid = lax.axis_index("c")
    sid = lax.axis_index("s")
    rpt = NP // NT
    bufs = rest[:NBUF]
    gsems = rest[NBUF:2 * NBUF]
    ssems = rest[2 * NBUF:]
    hme = h2.at[cid]

    for p in range(3):
        pltpu.sync_copy(hme.at[pl.ds(p * NP + sid * rpt, rpt)],
                        acc.at[pl.ds(sid * rpt, rpt)])
        plsc.subcore_barrier()

        def outer(o, carry):
            par = o % 2
            pltpu.sync_copy(src_i.at[p, sid, pl.ds(o * NBUF, NBUF)], srcv)
            pltpu.sync_copy(dst_i.at[p, sid, pl.ds(o * NBUF, NBUF)],
                            dstv.at[par])
            for i in range(NBUF):
                @pl.when(o > 0)
                def _drain():
                    pltpu.make_async_copy(
                        hme.at[pl.ds(0, CH)], bufs[i], ssems[i]).wait()

            gds = [pltpu.async_copy(hme.at[srcv.at[i]], bufs[i], gsems[i])
                   for i in range(NBUF)]
            for i in range(NBUF):
                gds[i].wait()
                pltpu.async_copy(bufs[i], acc.at[dstv.at[par, i]], ssems[i],
                                 add=True)
            return carry

        lax.fori_loop(0, GPP // NBUF, outer, 0)
        for i in range(NBUF):
            pltpu.make_async_copy(hme.at[pl.ds(0, CH)], bufs[i], ssems[i]).wait()
        plsc.subcore_barrier()
        pltpu.sync_copy(acc.at[pl.ds(sid * rpt, rpt)],
                        out_hbm.at[cid, p, pl.ds(sid * rpt, rpt)])
        plsc.subcore_barrier()


@functools.cache
def _sc_calls():
    mesh = plsc.VectorSubcoreMesh(core_axis_name="c", subcore_axis_name="s")
    cp = pltpu.CompilerParams(use_tc_tiling_on_sc=False)
    deg_call = pl.kernel(
        _sc_deg_body,
        out_type=jax.ShapeDtypeStruct((NC, R, 8), jnp.float32),
        mesh=mesh,
        compiler_params=cp,
        scratch_types=[
            pltpu.VMEM_SHARED((R, 8), jnp.float32),
            pltpu.VMEM((DIN, CH), jnp.int32),
            pltpu.VMEM((CH, 8), jnp.float32),
        ],
    )
    scat_call = pl.kernel(
        _sc_scatter_body,
        out_type=jax.ShapeDtypeStruct((NC, 3, NP, FH), jnp.float32),
        mesh=mesh,
        compiler_params=cp,
        scratch_types=(
            [pltpu.VMEM_SHARED((NP, FH), jnp.float32),
             pltpu.VMEM((NBUF, CH), jnp.int32),
             pltpu.VMEM((2, NBUF, CH), jnp.int32)]
            + [pltpu.VMEM((CH, FH), jnp.float32)] * NBUF
            + [pltpu.SemaphoreType.DMA] * (2 * NBUF)
        ),
    )
    return deg_call, scat_call


def _dinv_of(deg_ref):
    dtot = deg_ref[0, :, 0] + deg_ref[1, :, 0] + 1.0
    return lax.rsqrt(dtot)


def _tc_l1_body(x_ref, deg_ref, w_ref, out_ref):
    dinv = _dinv_of(deg_ref)
    h = jnp.dot(x_ref[...], w_ref[...], preferred_element_type=jnp.float32)
    hp = h * dinv[:, None]
    out_ref[0] = hp[:, :FH]
    out_ref[1] = hp[:, FH:]


def _tc_mid_body(acc_ref, deg_ref, w_ref, b_ref, out_ref):
    dinv = _dinv_of(deg_ref)
    full = jnp.concatenate([acc_ref[0, 0], acc_ref[1, 0]], axis=1)
    z = jnp.maximum(full * dinv[:, None] + b_ref[...], 0.0)
    h = jnp.dot(z, w_ref[...], preferred_element_type=jnp.float32)
    hp = h * dinv[:, None]
    out_ref[0] = hp[:, :FH]
    out_ref[1] = hp[:, FH:]


def _tc_pool_body(acc_ref, deg_ref, b_ref, seg_ref,
                  pooled_ref, cnt_ref, pacc, cacc):
    i = pl.program_id(0)

    @pl.when(i == 0)
    def _init():
        pacc[...] = jnp.zeros_like(pacc)
        cacc[...] = jnp.zeros_like(cacc)

    dinv = _dinv_of(deg_ref)
    full = jnp.concatenate([acc_ref[0, 0], acc_ref[1, 0]], axis=1)
    outc = full * dinv[:, None] + b_ref[...]
    seg = seg_ref[:, 0]
    cols = lax.broadcasted_iota(jnp.int32, (BR, 64), 1)
    p = (seg[:, None] == cols).astype(jnp.float32)
    pacc[...] += lax.dot_general(p, outc, (((0,), (0,)), ((), ())),
                                 preferred_element_type=jnp.float32)
    cacc[...] += jnp.broadcast_to(jnp.sum(p, axis=0)[:, None], (64, F))

    @pl.when(i == RB - 1)
    def _fin():
        pooled_ref[...] = pacc[...]
        cnt_ref[...] = cacc[...]


def _tc_head_body(pooled_ref, cnt_ref, l0w_ref, l0b_ref, lw_ref, lb_ref,
                  e0_ref, e1_ref, e2_ref, corr_ref, sp_ref, sn_ref, cs_ref):
    mean = pooled_ref[...] / jnp.maximum(cnt_ref[...], 1.0)
    e = jnp.dot(mean, l0w_ref[...], preferred_element_type=jnp.float32) + l0b_ref[...]
    e0 = e[0:16]
    e1 = e[16:32]
    e2 = e[32:48]
    e0_ref[...] = e0
    e1_ref[...] = e1
    e2_ref[...] = e2
    dp = jnp.sqrt(jnp.sum((e0 - e1 + 1e-6) ** 2, axis=1, keepdims=True))
    dn = jnp.sqrt(jnp.sum((e0 - e2 + 1e-6) ** 2, axis=1, keepdims=True))
    lw = lw_ref[...]
    y1 = (jnp.sum(e0 * lw[:, :64], axis=1, keepdims=True)
          + jnp.sum(e1 * lw[:, 64:], axis=1, keepdims=True) + lb_ref[...])
    y2 = (jnp.sum(e0 * lw[:, :64], axis=1, keepdims=True)
          + jnp.sum(e2 * lw[:, 64:], axis=1, keepdims=True) + lb_ref[...])
    sp = jax.nn.sigmoid(y1)
    sn = jax.nn.sigmoid(y2)
    sp_ref[...] = sp
    sn_ref[...] = sn
    corr_ref[...] = jnp.sum((dn - dp > 0).astype(jnp.int32), axis=(0, 1),
                            keepdims=True)
    cs_ref[...] = jnp.sum((sp - sn > 0).astype(jnp.int32), axis=(0, 1),
                          keepdims=True)


def _tc_l1(x, degacc, w0):
    return pl.pallas_call(
        _tc_l1_body,
        grid=(RB,),
        in_specs=[
            pl.BlockSpec((BR, F), lambda i: (i, 0)),
            pl.BlockSpec((NC, BR, 8), lambda i: (0, i, 0)),
            pl.BlockSpec((F, F), lambda i: (0, 0)),
        ],
        out_specs=pl.BlockSpec((NC, BR, FH), lambda i: (0, i, 0)),
        out_shape=jax.ShapeDtypeStruct((NC, R, FH), jnp.float32),
    )(x, degacc, w0)


def _tc_mid(acc, degacc, w, b):
    return pl.pallas_call(
        _tc_mid_body,
        grid=(RB,),
        in_specs=[
            pl.BlockSpec((NC, 1, BR, FH), lambda i: (0, i // 4, i % 4, 0)),
            pl.BlockSpec((NC, BR, 8), lambda i: (0, i, 0)),
            pl.BlockSpec((F, F), lambda i: (0, 0)),
            pl.BlockSpec((1, F), lambda i: (0, 0)),
        ],
        out_specs=pl.BlockSpec((NC, BR, FH), lambda i: (0, i, 0)),
        out_shape=jax.ShapeDtypeStruct((NC, R, FH), jnp.float32),
    )(acc, degacc, w, b)


def _tc_pool(acc, degacc, b, seg8):
    return pl.pallas_call(
        _tc_pool_body,
        grid=(RB,),
        in_specs=[
            pl.BlockSpec((NC, 1, BR, FH), lambda i: (0, i // 4, i % 4, 0)),
            pl.BlockSpec((NC, BR, 8), lambda i: (0, i, 0)),
            pl.BlockSpec((1, F), lambda i: (0, 0)),
            pl.BlockSpec((BR, 8), lambda i: (i, 0)),
        ],
        out_specs=[
            pl.BlockSpec((64, F), lambda i: (0, 0)),
            pl.BlockSpec((64, F), lambda i: (0, 0)),
        ],
        out_shape=[
            jax.ShapeDtypeStruct((64, F), jnp.float32),
            jax.ShapeDtypeStruct((64, F), jnp.float32),
        ],
        scratch_shapes=[
            pltpu.VMEM((64, F), jnp.float32),
            pltpu.VMEM((64, F), jnp.float32),
        ],
    )(acc, degacc, b, seg8)


def _tc_head(pooled, cnt, l0w, l0b, lw, lb):
    return pl.pallas_call(
        _tc_head_body,
        out_shape=[
            jax.ShapeDtypeStruct((NG, 64), jnp.float32),
            jax.ShapeDtypeStruct((NG, 64), jnp.float32),
            jax.ShapeDtypeStruct((NG, 64), jnp.float32),
            jax.ShapeDtypeStruct((1, 1), jnp.int32),
            jax.ShapeDtypeStruct((NG, 1), jnp.float32),
            jax.ShapeDtypeStruct((NG, 1), jnp.float32),
            jax.ShapeDtypeStruct((1, 1), jnp.int32),
        ],
    )(pooled, cnt, l0w, l0b, lw, lb)


def kernel(x0, edge_index0, batch0, x1, edge_index1, batch1,
           x2, edge_index2, batch2, params):
    xs = (x0, x1, x2)
    eis = (edge_index0, edge_index1, edge_index2)
    bs = (batch0, batch1, batch2)

    zpad = jnp.zeros((NP - N, F), jnp.float32)
    x_all = jnp.concatenate([jnp.concatenate([x, zpad]) for x in xs])
    epad = EPP - E
    srcs, dsts, dstg = [], [], []
    for p in range(3):
        s = jnp.concatenate([eis[p][0] + p * NP, jnp.zeros((epad,), jnp.int32)])
        d = jnp.concatenate([eis[p][1], jnp.full((epad,), DUMMY, jnp.int32)])
        srcs.append(s.reshape(NT, GPP, CH))
        dsts.append(d.reshape(NT, GPP, CH))
        dstg.append(d + p * NP)
    src_i = jnp.stack(srcs)
    dst_i = jnp.stack(dsts)
    dst4 = jnp.concatenate(dstg).reshape(NC, NT, 3 * EPP // (NC * NT * CH), CH)
    segpad = jnp.full((NP - N,), 48, jnp.int32)
    seg = jnp.concatenate(
        [jnp.concatenate([bs[p] + NG * p, segpad]) for p in range(3)])
    seg8 = jnp.broadcast_to(seg[:, None], (R, 8))

    zeros8 = jnp.zeros((R, 8), jnp.float32)
    ones8 = jnp.ones((CH, 8), jnp.float32)

    w = params["conv_W"]
    cb = params["conv_b"]
    b0, b1, b2 = (cb[i].reshape(1, F) for i in range(3))

    deg_call, scat_call = _sc_calls()
    degacc = deg_call(dst4, zeros8, ones8)
    h1p = _tc_l1(x_all, degacc, w[0])
    a1 = scat_call(h1p, src_i, dst_i)
    h2p = _tc_mid(a1, degacc, w[1], b0)
    a2 = scat_call(h2p, src_i, dst_i)
    h3p = _tc_mid(a2, degacc, w[2], b1)
    a3 = scat_call(h3p, src_i, dst_i)
    pooled, cnt = _tc_pool(a3, degacc, b2, seg8)
    e0, e1, e2, corr, sp, sn, cs = _tc_head(
        pooled, cnt, params["lin0_W"], params["lin0_b"].reshape(1, 64),
        params["lin_W"].reshape(1, F), params["lin_b"].reshape(1, 1))
    return (e0, e1, e2, corr.reshape(1), sp, sn, cs.reshape(1))


# speedup vs baseline: 2.7022x; 1.7505x over previous
import functools

import jax
import jax.numpy as jnp
from jax import lax
from jax.experimental import pallas as pl
from jax.experimental.pallas import tpu as pltpu
from jax.experimental.pallas import tpu_sc as plsc

N = 10000
E = 320000
F = 128
FH = 64
NG = 16
NP = 10112
DUMMY = 10016
R = 3 * NP
EPP = 327680
CH = 128
NT = 16
NC = 2
GPP = 160
NBUF = 4
DIN = 24
BR = 2528
RB = R // BR


def _sc_deg_body(dst4, zeros8, ones8, deg_out, acc, idxv, onesv):
    cid = lax.axis_index("c")
    sid = lax.axis_index("s")
    rpt = R // NT
    pltpu.sync_copy(zeros8.at[pl.ds(sid * rpt, rpt)], acc.at[pl.ds(sid * rpt, rpt)])
    pltpu.sync_copy(ones8, onesv)
    plsc.subcore_barrier()

    def outer(o, carry):
        pltpu.sync_copy(dst4.at[cid, sid, pl.ds(o * DIN, DIN)], idxv)
        for j in range(DIN):
            pltpu.sync_copy(onesv, acc.at[idxv.at[j]], add=True)
        return carry

    lax.fori_loop(0, EPP * 3 // (NC * NT * CH * DIN), outer, 0)
    plsc.subcore_barrier()
    pltpu.sync_copy(acc.at[pl.ds(sid * rpt, rpt)],
                    deg_out.at[cid, pl.ds(sid * rpt, rpt)])


def _sc_scatter_body(h2, src_i, dst_i, out_hbm, acc, hsp, srcv, dstv, *rest):
    cid = lax.axis_index("c")
    sid = lax.axis_index("s")
    rpt = NP // NT
    bufs = rest[:NBUF]
    gsems = rest[NBUF:2 * NBUF]
    ssems = rest[2 * NBUF:]
    hme = h2.at[cid]

    for p in range(3):
        pltpu.sync_copy(hme.at[pl.ds(p * NP + sid * rpt, rpt)],
                        hsp.at[pl.ds(sid * rpt, rpt)])
        pltpu.sync_copy(hme.at[pl.ds(p * NP + sid * rpt, rpt)],
                        acc.at[pl.ds(sid * rpt, rpt)])
        plsc.subcore_barrier()

        def outer(o, carry):
            par = o % 2
            pltpu.sync_copy(src_i.at[p, sid, pl.ds(o * NBUF, NBUF)], srcv)
            pltpu.sync_copy(dst_i.at[p, sid, pl.ds(o * NBUF, NBUF)],
                            dstv.at[par])
            for i in range(NBUF):
                @pl.when(o > 0)
                def _drain():
                    pltpu.make_async_copy(
                        hme.at[pl.ds(0, CH)], bufs[i], ssems[i]).wait()

            gds = [pltpu.async_copy(hsp.at[srcv.at[i]], bufs[i], gsems[i])
                   for i in range(NBUF)]
            for i in range(NBUF):
                gds[i].wait()
                pltpu.async_copy(bufs[i], acc.at[dstv.at[par, i]], ssems[i],
                                 add=True)
            return carry

        lax.fori_loop(0, GPP // NBUF, outer, 0)
        for i in range(NBUF):
            pltpu.make_async_copy(hme.at[pl.ds(0, CH)], bufs[i], ssems[i]).wait()
        plsc.subcore_barrier()
        pltpu.sync_copy(acc.at[pl.ds(sid * rpt, rpt)],
                        out_hbm.at[cid, p, pl.ds(sid * rpt, rpt)])
        plsc.subcore_barrier()


@functools.cache
def _sc_calls():
    mesh = plsc.VectorSubcoreMesh(core_axis_name="c", subcore_axis_name="s")
    cp = pltpu.CompilerParams(use_tc_tiling_on_sc=False)
    deg_call = pl.kernel(
        _sc_deg_body,
        out_type=jax.ShapeDtypeStruct((NC, R, 8), jnp.float32),
        mesh=mesh,
        compiler_params=cp,
        scratch_types=[
            pltpu.VMEM_SHARED((R, 8), jnp.float32),
            pltpu.VMEM((DIN, CH), jnp.int32),
            pltpu.VMEM((CH, 8), jnp.float32),
        ],
    )
    scat_call = pl.kernel(
        _sc_scatter_body,
        out_type=jax.ShapeDtypeStruct((NC, 3, NP, FH), jnp.float32),
        mesh=mesh,
        compiler_params=cp,
        scratch_types=(
            [pltpu.VMEM_SHARED((NP, FH), jnp.float32),
             pltpu.VMEM_SHARED((NP, FH), jnp.float32),
             pltpu.VMEM((NBUF, CH), jnp.int32),
             pltpu.VMEM((2, NBUF, CH), jnp.int32)]
            + [pltpu.VMEM((CH, FH), jnp.float32)] * NBUF
            + [pltpu.SemaphoreType.DMA] * (2 * NBUF)
        ),
    )
    return deg_call, scat_call


def _dinv_of(deg_ref):
    dtot = deg_ref[0, :, 0] + deg_ref[1, :, 0] + 1.0
    return lax.rsqrt(dtot)


def _tc_l1_body(x_ref, deg_ref, w_ref, out_ref):
    dinv = _dinv_of(deg_ref)
    h = jnp.dot(x_ref[...], w_ref[...], preferred_element_type=jnp.float32)
    hp = h * dinv[:, None]
    out_ref[0] = hp[:, :FH]
    out_ref[1] = hp[:, FH:]


def _tc_mid_body(acc_ref, deg_ref, w_ref, b_ref, out_ref):
    dinv = _dinv_of(deg_ref)
    full = jnp.concatenate([acc_ref[0, 0], acc_ref[1, 0]], axis=1)
    z = jnp.maximum(full * dinv[:, None] + b_ref[...], 0.0)
    h = jnp.dot(z, w_ref[...], preferred_element_type=jnp.float32)
    hp = h * dinv[:, None]
    out_ref[0] = hp[:, :FH]
    out_ref[1] = hp[:, FH:]


def _tc_pool_body(acc_ref, deg_ref, b_ref, seg_ref,
                  pooled_ref, cnt_ref, pacc, cacc):
    i = pl.program_id(0)

    @pl.when(i == 0)
    def _init():
        pacc[...] = jnp.zeros_like(pacc)
        cacc[...] = jnp.zeros_like(cacc)

    dinv = _dinv_of(deg_ref)
    full = jnp.concatenate([acc_ref[0, 0], acc_ref[1, 0]], axis=1)
    outc = full * dinv[:, None] + b_ref[...]
    seg = seg_ref[:, 0]
    cols = lax.broadcasted_iota(jnp.int32, (BR, 64), 1)
    p = (seg[:, None] == cols).astype(jnp.float32)
    pacc[...] += lax.dot_general(p, outc, (((0,), (0,)), ((), ())),
                                 preferred_element_type=jnp.float32)
    cacc[...] += jnp.broadcast_to(jnp.sum(p, axis=0)[:, None], (64, F))

    @pl.when(i == RB - 1)
    def _fin():
        pooled_ref[...] = pacc[...]
        cnt_ref[...] = cacc[...]


def _tc_head_body(pooled_ref, cnt_ref, l0w_ref, l0b_ref, lw_ref, lb_ref,
                  e0_ref, e1_ref, e2_ref, corr_ref, sp_ref, sn_ref, cs_ref):
    mean = pooled_ref[...] / jnp.maximum(cnt_ref[...], 1.0)
    e = jnp.dot(mean, l0w_ref[...], preferred_element_type=jnp.float32) + l0b_ref[...]
    e0 = e[0:16]
    e1 = e[16:32]
    e2 = e[32:48]
    e0_ref[...] = e0
    e1_ref[...] = e1
    e2_ref[...] = e2
    dp = jnp.sqrt(jnp.sum((e0 - e1 + 1e-6) ** 2, axis=1, keepdims=True))
    dn = jnp.sqrt(jnp.sum((e0 - e2 + 1e-6) ** 2, axis=1, keepdims=True))
    lw = lw_ref[...]
    y1 = (jnp.sum(e0 * lw[:, :64], axis=1, keepdims=True)
          + jnp.sum(e1 * lw[:, 64:], axis=1, keepdims=True) + lb_ref[...])
    y2 = (jnp.sum(e0 * lw[:, :64], axis=1, keepdims=True)
          + jnp.sum(e2 * lw[:, 64:], axis=1, keepdims=True) + lb_ref[...])
    sp = jax.nn.sigmoid(y1)
    sn = jax.nn.sigmoid(y2)
    sp_ref[...] = sp
    sn_ref[...] = sn
    corr_ref[...] = jnp.sum((dn - dp > 0).astype(jnp.int32), axis=(0, 1),
                            keepdims=True)
    cs_ref[...] = jnp.sum((sp - sn > 0).astype(jnp.int32), axis=(0, 1),
                          keepdims=True)


def _tc_l1(x, degacc, w0):
    return pl.pallas_call(
        _tc_l1_body,
        grid=(RB,),
        in_specs=[
            pl.BlockSpec((BR, F), lambda i: (i, 0)),
            pl.BlockSpec((NC, BR, 8), lambda i: (0, i, 0)),
            pl.BlockSpec((F, F), lambda i: (0, 0)),
        ],
        out_specs=pl.BlockSpec((NC, BR, FH), lambda i: (0, i, 0)),
        out_shape=jax.ShapeDtypeStruct((NC, R, FH), jnp.float32),
    )(x, degacc, w0)


def _tc_mid(acc, degacc, w, b):
    return pl.pallas_call(
        _tc_mid_body,
        grid=(RB,),
        in_specs=[
            pl.BlockSpec((NC, 1, BR, FH), lambda i: (0, i // 4, i % 4, 0)),
            pl.BlockSpec((NC, BR, 8), lambda i: (0, i, 0)),
            pl.BlockSpec((F, F), lambda i: (0, 0)),
            pl.BlockSpec((1, F), lambda i: (0, 0)),
        ],
        out_specs=pl.BlockSpec((NC, BR, FH), lambda i: (0, i, 0)),
        out_shape=jax.ShapeDtypeStruct((NC, R, FH), jnp.float32),
    )(acc, degacc, w, b)


def _tc_pool(acc, degacc, b, seg8):
    return pl.pallas_call(
        _tc_pool_body,
        grid=(RB,),
        in_specs=[
            pl.BlockSpec((NC, 1, BR, FH), lambda i: (0, i // 4, i % 4, 0)),
            pl.BlockSpec((NC, BR, 8), lambda i: (0, i, 0)),
            pl.BlockSpec((1, F), lambda i: (0, 0)),
            pl.BlockSpec((BR, 8), lambda i: (i, 0)),
        ],
        out_specs=[
            pl.BlockSpec((64, F), lambda i: (0, 0)),
            pl.BlockSpec((64, F), lambda i: (0, 0)),
        ],
        out_shape=[
            jax.ShapeDtypeStruct((64, F), jnp.float32),
            jax.ShapeDtypeStruct((64, F), jnp.float32),
        ],
        scratch_shapes=[
            pltpu.VMEM((64, F), jnp.float32),
            pltpu.VMEM((64, F), jnp.float32),
        ],
    )(acc, degacc, b, seg8)


def _tc_head(pooled, cnt, l0w, l0b, lw, lb):
    return pl.pallas_call(
        _tc_head_body,
        out_shape=[
            jax.ShapeDtypeStruct((NG, 64), jnp.float32),
            jax.ShapeDtypeStruct((NG, 64), jnp.float32),
            jax.ShapeDtypeStruct((NG, 64), jnp.float32),
            jax.ShapeDtypeStruct((1, 1), jnp.int32),
            jax.ShapeDtypeStruct((NG, 1), jnp.float32),
            jax.ShapeDtypeStruct((NG, 1), jnp.float32),
            jax.ShapeDtypeStruct((1, 1), jnp.int32),
        ],
    )(pooled, cnt, l0w, l0b, lw, lb)


def kernel(x0, edge_index0, batch0, x1, edge_index1, batch1,
           x2, edge_index2, batch2, params):
    xs = (x0, x1, x2)
    eis = (edge_index0, edge_index1, edge_index2)
    bs = (batch0, batch1, batch2)

    zpad = jnp.zeros((NP - N, F), jnp.float32)
    x_all = jnp.concatenate([jnp.concatenate([x, zpad]) for x in xs])
    epad = EPP - E
    srcs, dsts, dstg = [], [], []
    for p in range(3):
        s = jnp.concatenate([eis[p][0], jnp.zeros((epad,), jnp.int32)])
        d = jnp.concatenate([eis[p][1], jnp.full((epad,), DUMMY, jnp.int32)])
        srcs.append(s.reshape(NT, GPP, CH))
        dsts.append(d.reshape(NT, GPP, CH))
        dstg.append(d + p * NP)
    src_i = jnp.stack(srcs)
    dst_i = jnp.stack(dsts)
    dst4 = jnp.concatenate(dstg).reshape(NC, NT, 3 * EPP // (NC * NT * CH), CH)
    segpad = jnp.full((NP - N,), 48, jnp.int32)
    seg = jnp.concatenate(
        [jnp.concatenate([bs[p] + NG * p, segpad]) for p in range(3)])
    seg8 = jnp.broadcast_to(seg[:, None], (R, 8))

    zeros8 = jnp.zeros((R, 8), jnp.float32)
    ones8 = jnp.ones((CH, 8), jnp.float32)

    w = params["conv_W"]
    cb = params["conv_b"]
    b0, b1, b2 = (cb[i].reshape(1, F) for i in range(3))

    deg_call, scat_call = _sc_calls()
    degacc = deg_call(dst4, zeros8, ones8)
    h1p = _tc_l1(x_all, degacc, w[0])
    a1 = scat_call(h1p, src_i, dst_i)
    h2p = _tc_mid(a1, degacc, w[1], b0)
    a2 = scat_call(h2p, src_i, dst_i)
    h3p = _tc_mid(a2, degacc, w[2], b1)
    a3 = scat_call(h3p, src_i, dst_i)
    pooled, cnt = _tc_pool(a3, degacc, b2, seg8)
    e0, e1, e2, corr, sp, sn, cs = _tc_head(
        pooled, cnt, params["lin0_W"], params["lin0_b"].reshape(1, 64),
        params["lin_W"].reshape(1, F), params["lin_b"].reshape(1, 1))
    return (e0, e1, e2, corr.reshape(1), sp, sn, cs.reshape(1))
